# Initial kernel scaffold; baseline (speedup 1.0000x reference)
import functools
import jax
import jax.numpy as jnp
from jax import lax
from jax.experimental import pallas as pl
from jax.experimental.pallas import tpu as pltpu
from jax.experimental.pallas import tpu_sc as plsc

N = 10000
E = 320000
D = 128
HID = 128
SEQ = 3
CHOL_N = 512
CHOL_ELEMS = CHOL_N * (CHOL_N + 1) // 2
CLIP = 100000.0

NC = 2
NS = 16
NW = NC * NS

NPAD = 10240
E_EXT = E + N
E_PAD = 330240
CHUNK = 80
ROWS_PER_W = E_PAD // NW // CHUNK
NROWS = E_PAD // CHUNK
STRIPE = NPAD // NS


def _nan2num(t):
  return jnp.nan_to_num(t, nan=0.0, posinf=CLIP, neginf=-CLIP)


def _sc_deg_body(c2d, w2d, out, cbuf, wbuf, zbuf, deg0, deg1, deg2):
  cid = lax.axis_index("c")
  sid = lax.axis_index("s")
  wid = cid * NS + sid
  degs = [deg0, deg1, deg2]

  for i in range(STRIPE // 16):
    zbuf[pl.ds(i * 16, 16)] = jnp.zeros((16,), jnp.float32)
  for dref in degs:
    pltpu.sync_copy(zbuf, dref.at[pl.ds(sid * STRIPE, STRIPE)])
  plsc.subcore_barrier()

  pltpu.sync_copy(c2d.at[pl.ds(wid * ROWS_PER_W, ROWS_PER_W)], cbuf)

  for t in range(SEQ):
    pltpu.sync_copy(
        w2d.at[pl.ds(t * NROWS + wid * ROWS_PER_W, ROWS_PER_W)], wbuf)

    def chunk(k, carry):
      pltpu.sync_copy(wbuf.at[k], degs[t].at[cbuf.at[k]], add=True)
      return carry

    lax.fori_loop(0, ROWS_PER_W, chunk, 0)

  plsc.subcore_barrier()
  for t in range(SEQ):
    pltpu.sync_copy(
        degs[t].at[pl.ds(sid * STRIPE, STRIPE)],
        out.at[cid * SEQ + t, pl.ds(sid * STRIPE, STRIPE)])


def _sc_deg(c2d, w2d):
  kfn = pl.kernel(
      _sc_deg_body,
      out_type=jax.ShapeDtypeStruct((NC * SEQ, NPAD), jnp.float32),
      mesh=plsc.VectorSubcoreMesh(
          core_axis_name="c", subcore_axis_name="s",
          num_cores=NC, num_subcores=NS),
      scratch_types=[
          pltpu.VMEM((ROWS_PER_W, CHUNK), jnp.int32),
          pltpu.VMEM((ROWS_PER_W, CHUNK), jnp.float32),
          pltpu.VMEM((STRIPE,), jnp.float32),
          pltpu.VMEM_SHARED((NPAD,), jnp.float32),
          pltpu.VMEM_SHARED((NPAD,), jnp.float32),
          pltpu.VMEM_SHARED((NPAD,), jnp.float32),
      ],
  )
  return kfn(c2d, w2d)


def _sc_msg_body(t0, t1, t2, r2d, c2d, w2d, dinv_hbm, out,
                 rbuf, cbuf, wbuf, dinvb, normb, rb0, rb1, rb2, zrow, acc,
                 gs0, gs1, gs2, ss0, ss1, ss2):
  cid = lax.axis_index("c")
  sid = lax.axis_index("s")
  wid = cid * NS + sid
  tables = [t0, t1, t2]
  rowbufs = [rb0, rb1, rb2]
  gsems = [gs0, gs1, gs2]
  ssems = [ss0, ss1, ss2]

  pltpu.sync_copy(r2d.at[pl.ds(wid * ROWS_PER_W, ROWS_PER_W)], rbuf)
  pltpu.sync_copy(c2d.at[pl.ds(wid * ROWS_PER_W, ROWS_PER_W)], cbuf)
  pltpu.sync_copy(dinv_hbm, dinvb)

  def zrow_init(i, carry):
    for j in range(D // 16):
      zrow[i, pl.ds(j * 16, 16)] = jnp.zeros((16,), jnp.float32)
    return carry
  lax.fori_loop(0, CHUNK, zrow_init, 0)

  for z in range(STRIPE // CHUNK):
    pltpu.sync_copy(zrow, acc.at[pl.ds(sid * STRIPE + z * CHUNK, CHUNK)])
  plsc.subcore_barrier()

  nchunks = ROWS_PER_W
  nouter = nchunks // 3

  for t in range(SEQ):
    table = tables[t]
    pltpu.sync_copy(
        w2d.at[pl.ds(t * NROWS + wid * ROWS_PER_W, ROWS_PER_W)], wbuf)

    for b in range(3):
      pltpu.async_copy(table.at[rbuf.at[b]], rowbufs[b], gsems[b])

    def outer(m, carry):
      for b in range(3):
        k = 3 * m + b
        rowb = rowbufs[b]
        pltpu.make_async_copy(table.at[rbuf.at[k]], rowb, gsems[b]).wait()

        toff = t * NPAD
        for j in range(CHUNK // 16):
          sl = pl.ds(j * 16, 16)
          ridx = rbuf[k, sl]
          cidx = cbuf[k, sl]
          wv = wbuf[k, sl]
          dr = plsc.load_gather(dinvb, [ridx + toff])
          dc = plsc.load_gather(dinvb, [cidx + toff])
          normb[sl] = dr * wv * dc

        def scale(q, carry2):
          for u in range(4):
            e = q * 4 + u
            s = normb[e]
            for j in range(D // 16):
              sl = pl.ds(j * 16, 16)
              rowb[e, sl] = rowb[e, sl] * s
          return carry2
        lax.fori_loop(0, CHUNK // 4, scale, 0)

        pltpu.async_copy(rowb, acc.at[cbuf.at[k]], ssems[b], add=True)

        @pl.when(k < nchunks - 3)
        def _():
          pltpu.make_async_copy(rowb, acc.at[cbuf.at[k]], ssems[b]).wait()
          pltpu.async_copy(table.at[rbuf.at[k + 3]], rowb, gsems[b])
      return carry

    lax.fori_loop(0, nouter, outer, 0)

    for b in range(3):
      k = nchunks - 3 + b
      pltpu.make_async_copy(rowbufs[b], acc.at[cbuf.at[k]], ssems[b]).wait()
    plsc.subcore_barrier()

    pltpu.sync_copy(
        acc.at[pl.ds(sid * STRIPE, STRIPE)],
        out.at[cid * SEQ + t, pl.ds(sid * STRIPE, STRIPE)])
    if t < SEQ - 1:
      for z in range(STRIPE // CHUNK):
        pltpu.sync_copy(zrow, acc.at[pl.ds(sid * STRIPE + z * CHUNK, CHUNK)])
      plsc.subcore_barrier()


def _sc_msg(t0, t1, t2, r2d, c2d, w2d, dinv_flat):
  kfn = pl.kernel(
      _sc_msg_body,
      out_type=jax.ShapeDtypeStruct((NC * SEQ, NPAD, D), jnp.float32),
      mesh=plsc.VectorSubcoreMesh(
          core_axis_name="c", subcore_axis_name="s",
          num_cores=NC, num_subcores=NS),
      scratch_types=[
          pltpu.VMEM((ROWS_PER_W, CHUNK), jnp.int32),
          pltpu.VMEM((ROWS_PER_W, CHUNK), jnp.int32),
          pltpu.VMEM((ROWS_PER_W, CHUNK), jnp.float32),
          pltpu.VMEM((SEQ * NPAD,), jnp.float32),
          pltpu.VMEM((CHUNK,), jnp.float32),
          pltpu.VMEM((CHUNK, D), jnp.float32),
          pltpu.VMEM((CHUNK, D), jnp.float32),
          pltpu.VMEM((CHUNK, D), jnp.float32),
          pltpu.VMEM((CHUNK, D), jnp.float32),
          pltpu.VMEM_SHARED((NPAD, D), jnp.float32),
          pltpu.SemaphoreType.DMA,
          pltpu.SemaphoreType.DMA,
          pltpu.SemaphoreType.DMA,
          pltpu.SemaphoreType.DMA,
          pltpu.SemaphoreType.DMA,
          pltpu.SemaphoreType.DMA,
      ],
  )
  return kfn(t0, t1, t2, r2d, c2d, w2d, dinv_flat)


def _sc_lbuild_body(chol_hbm, out, slab, rowb):
  cid = lax.axis_index("c")
  sid = lax.axis_index("s")
  wid = cid * NS + sid
  iota = lax.broadcasted_iota(jnp.int32, (16,), 0)
  for m in range(CHOL_N // NW):
    i = wid * (CHOL_N // NW) + m
    off = (i * (i + 1)) // 2
    off_al = pl.multiple_of((off // 8) * 8, 8)
    sh = off - off_al
    pltpu.sync_copy(chol_hbm.at[pl.ds(off_al, CHOL_N + 24)], slab)
    for j in range(CHOL_N // 16):
      v = slab[pl.ds(sh + j * 16, 16)]
      pos = j * 16 + iota
      v = jnp.where(pos <= i, v, jnp.zeros((16,), jnp.float32))
      rowb[pl.ds(j * 16, 16)] = v
    pltpu.sync_copy(rowb, out.at[i])


def _sc_lbuild(chol_pad):
  kfn = pl.kernel(
      _sc_lbuild_body,
      out_type=jax.ShapeDtypeStruct((CHOL_N, CHOL_N), jnp.float32),
      mesh=plsc.VectorSubcoreMesh(
          core_axis_name="c", subcore_axis_name="s",
          num_cores=NC, num_subcores=NS),
      scratch_types=[
          pltpu.VMEM((CHOL_N + 24,), jnp.float32),
          pltpu.VMEM((CHOL_N,), jnp.float32),
      ],
  )
  return kfn(chol_pad)


def _tc_dinv_body(degp_ref, dinv_ref):
  dsum = degp_ref[0] + degp_ref[1]
  dinv_ref[...] = jnp.where(
      dsum > 0, lax.rsqrt(jnp.where(dsum > 0, dsum, 1.0)), 0.0)


def _tc_dinv(degp):
  degp = degp.reshape(NC, SEQ, NPAD)
  return pl.pallas_call(
      _tc_dinv_body,
      out_shape=jax.ShapeDtypeStruct((SEQ, NPAD), jnp.float32),
  )(degp)


_MM_BLK = 1200


def _tc_mm_body(x_ref, w_ref, o_ref):
  o_ref[...] = jnp.dot(x_ref[...], w_ref[...],
                       preferred_element_type=jnp.float32)


def _tc_mm(x2d, w):
  rows = x2d.shape[0]
  return pl.pallas_call(
      _tc_mm_body,
      grid=(rows // _MM_BLK,),
      in_specs=[
          pl.BlockSpec((_MM_BLK, D), lambda i: (i, 0)),
          pl.BlockSpec((D, D), lambda i: (0, 0)),
      ],
      out_specs=pl.BlockSpec((_MM_BLK, D), lambda i: (i, 0)),
      out_shape=jax.ShapeDtypeStruct((rows, D), jnp.float32),
  )(x2d, w)


_RB = 400


def _tc_fuse_mm_body(p0_ref, p1_ref, b_ref, w_ref, o_ref):
  h = jax.nn.relu(_nan2num(p0_ref[0] + p1_ref[0] + b_ref[...]))
  o_ref[0] = jnp.dot(h, w_ref[...], preferred_element_type=jnp.float32)


def _tc_fuse_mm(p0, p1, b, w):
  b2 = b.reshape(1, D)
  return pl.pallas_call(
      _tc_fuse_mm_body,
      grid=(SEQ, N // _RB),
      in_specs=[
          pl.BlockSpec((1, _RB, D), lambda t, i: (t, i, 0)),
          pl.BlockSpec((1, _RB, D), lambda t, i: (t, i, 0)),
          pl.BlockSpec((1, D), lambda t, i: (0, 0)),
          pl.BlockSpec((D, D), lambda t, i: (0, 0)),
      ],
      out_specs=pl.BlockSpec((1, _RB, D), lambda t, i: (t, i, 0)),
      out_shape=jax.ShapeDtypeStruct((SEQ, N, D), jnp.float32),
  )(p0, p1, b2, w)


def _tc_emb_body(p0_ref, p1_ref, b_ref, o_ref):
  i = pl.program_id(1)
  h = jax.nn.relu(_nan2num(p0_ref[0] + p1_ref[0] + b_ref[...]))
  s = jnp.sum(h, axis=0, keepdims=True)

  @pl.when(i == 0)
  def _():
    o_ref[...] = jnp.zeros_like(o_ref)

  o_ref[...] += s

  @pl.when(i == N // _RB - 1)
  def _():
    o_ref[...] = _nan2num(o_ref[...] / float(N))


def _tc_emb(p0, p1, b):
  b2 = b.reshape(1, D)
  return pl.pallas_call(
      _tc_emb_body,
      grid=(SEQ, N // _RB),
      in_specs=[
          pl.BlockSpec((1, _RB, D), lambda t, i: (t, i, 0)),
          pl.BlockSpec((1, _RB, D), lambda t, i: (t, i, 0)),
          pl.BlockSpec((1, D), lambda t, i: (0, 0)),
      ],
      out_specs=pl.BlockSpec((1, D), lambda t, i: (t, 0)),
      out_shape=jax.ShapeDtypeStruct((SEQ, D), jnp.float32),
  )(p0, p1, b2)


def _tc_lstm_body(emb_ref, wih_ref, whh_ref, bih_ref, bhh_ref, o_ref):
  h = jnp.zeros((1, HID), jnp.float32)
  c = jnp.zeros((1, HID), jnp.float32)
  wih = wih_ref[...]
  whh = whh_ref[...]
  bias = bih_ref[...] + bhh_ref[...]
  dn = (((1,), (1,)), ((), ()))
  for t in range(SEQ):
    xt = emb_ref[pl.ds(t, 1), :]
    gates = (lax.dot_general(xt, wih, dn, preferred_element_type=jnp.float32)
             + lax.dot_general(h, whh, dn, preferred_element_type=jnp.float32)
             + bias)
    ig = jax.nn.sigmoid(gates[:, 0:HID])
    fg = jax.nn.sigmoid(gates[:, HID:2 * HID])
    gg = jnp.tanh(gates[:, 2 * HID:3 * HID])
    og = jax.nn.sigmoid(gates[:, 3 * HID:4 * HID])
    c = fg * c + ig * gg
    h = og * jnp.tanh(c)
  o_ref[...] = _nan2num(h)


def _tc_lstm(emb, wih, whh, bih, bhh):
  return pl.pallas_call(
      _tc_lstm_body,
      out_shape=jax.ShapeDtypeStruct((1, HID), jnp.float32),
  )(emb, wih, whh, bih.reshape(1, 4 * HID), bhh.reshape(1, 4 * HID))


_CB = 2304


def _tc_chol_body(fh_ref, w_ref, b_ref, o_ref):
  o_ref[...] = _nan2num(
      jnp.dot(fh_ref[...], w_ref[...], preferred_element_type=jnp.float32)
      + b_ref[...])


def _tc_chol(fh, w_fc, b_fc):
  return pl.pallas_call(
      _tc_chol_body,
      grid=(CHOL_ELEMS // _CB,),
      in_specs=[
          pl.BlockSpec((1, HID), lambda i: (0, 0)),
          pl.BlockSpec((HID, _CB), lambda i: (0, i)),
          pl.BlockSpec((1, _CB), lambda i: (0, i)),
      ],
      out_specs=pl.BlockSpec((1, _CB), lambda i: (0, i)),
      out_shape=jax.ShapeDtypeStruct((1, CHOL_ELEMS), jnp.float32),
  )(fh, w_fc, b_fc.reshape(1, CHOL_ELEMS))


def _tc_llt_body(l_ref, o_ref):
  l = l_ref[...]
  o_ref[...] = _nan2num(
      lax.dot_general(l, l, (((1,), (1,)), ((), ())),
                      preferred_element_type=jnp.float32))


def _tc_llt(l):
  return pl.pallas_call(
      _tc_llt_body,
      out_shape=jax.ShapeDtypeStruct((CHOL_N, CHOL_N), jnp.float32),
  )(l)


def kernel(x, edge_index, edge_weight, W1, b1, W2, b2,
           W_ih, W_hh, b_ih, b_hh, W_fc, b_fc):
  row, col = edge_index[0], edge_index[1]

  loop = jnp.arange(N, dtype=row.dtype)
  padi = jnp.zeros((E_PAD - E_EXT,), row.dtype)
  r_ext = jnp.concatenate([row, loop, padi])
  c_ext = jnp.concatenate([col, loop, padi])
  w_ext = jnp.concatenate(
      [edge_weight,
       jnp.ones((SEQ, N), jnp.float32),
       jnp.zeros((SEQ, E_PAD - E_EXT), jnp.float32)], axis=1)

  r2d = r_ext.reshape(NROWS, CHUNK)
  c2d = c_ext.reshape(NROWS, CHUNK)
  w2d = w_ext.reshape(SEQ * NROWS, CHUNK)

  degp = _sc_deg(c2d, w2d)
  dinv = _tc_dinv(degp)
  dinv_flat = dinv.reshape(SEQ * NPAD)

  x2d = x.reshape(SEQ * N, D)
  xw1 = _tc_mm(x2d, W1)
  p1 = _sc_msg(xw1[0:N], xw1[N:2 * N], xw1[2 * N:3 * N],
               r2d, c2d, w2d, dinv_flat)
  p1 = p1.reshape(NC, SEQ, NPAD, D)

  xw2 = _tc_fuse_mm(p1[0], p1[1], b1, W2)
  p2 = _sc_msg(xw2[0], xw2[1], xw2[2], r2d, c2d, w2d, dinv_flat)
  p2 = p2.reshape(NC, SEQ, NPAD, D)

  emb = _tc_emb(p2[0], p2[1], b2)
  fh = _tc_lstm(emb, W_ih, W_hh, b_ih, b_hh)
  chol = _tc_chol(fh, W_fc, b_fc)

  chol_pad = jnp.concatenate(
      [chol.reshape(CHOL_ELEMS), jnp.zeros((CHOL_N,), jnp.float32)])
  l = _sc_lbuild(chol_pad)
  return _tc_llt(l)


# speedup vs baseline: 15.2505x; 15.2505x over previous
import jax
import jax.numpy as jnp
from jax import lax
from jax.experimental import pallas as pl
from jax.experimental.pallas import tpu as pltpu
from jax.experimental.pallas import tpu_sc as plsc

N = 10000
E = 320000
D = 128
HID = 128
SEQ = 3
CHOL_N = 512
CHOL_ELEMS = CHOL_N * (CHOL_N + 1) // 2
CLIP = 100000.0

NC = 2
NS = 16
NW = NC * NS

NPAD = 10240
NDEAD = 240
E_EXT = E + N
CHUNK = 64
ROWS_PER_W = 162
E_PAD = NW * ROWS_PER_W * CHUNK
NROWS = E_PAD // CHUNK
ROWS_PER_T = NROWS // NS
STRIPE = NPAD // NS

_SC_PARAMS = pltpu.CompilerParams(
    use_tc_tiling_on_sc=False, needs_layout_passes=False)


def _nan2num(t):
  return jnp.nan_to_num(t, nan=0.0, posinf=CLIP, neginf=-CLIP)


def _sc_mesh():
  return plsc.VectorSubcoreMesh(
      core_axis_name="c", subcore_axis_name="s",
      num_cores=NC, num_subcores=NS)


def _rsqrt16(d):
  half = d * 0.5
  ibits = plsc.bitcast(d, jnp.int32)
  y = plsc.bitcast(jnp.int32(0x5F3759DF) - lax.shift_right_logical(ibits, 1),
                   jnp.float32)
  for _ in range(3):
    y = y * (1.5 - half * y * y)
  return y


def _sc_norm_body(r2d, c2d, w2d, out,
                  call_buf, wall_buf, rown, cown, wown, dinvb, nout, zbuf,
                  deg_sh):
  cid = lax.axis_index("c")
  sid = lax.axis_index("s")
  wid = cid * NS + sid

  pltpu.sync_copy(c2d.at[pl.ds(sid * ROWS_PER_T, ROWS_PER_T)], call_buf)
  pltpu.sync_copy(r2d.at[pl.ds(wid * ROWS_PER_W, ROWS_PER_W)], rown)
  pltpu.sync_copy(c2d.at[pl.ds(wid * ROWS_PER_W, ROWS_PER_W)], cown)

  for i in range(STRIPE // 16):
    zbuf[pl.ds(i * 16, 16)] = jnp.zeros((16,), jnp.float32)

  for t in range(SEQ):
    pltpu.sync_copy(zbuf, deg_sh.at[pl.ds(sid * STRIPE, STRIPE)])
    plsc.subcore_barrier()

    pltpu.sync_copy(w2d.at[pl.ds(t * NROWS + sid * ROWS_PER_T, ROWS_PER_T)],
                    wall_buf)

    def dchunk(k, carry):
      pltpu.sync_copy(wall_buf.at[k], deg_sh.at[call_buf.at[k]], add=True)
      return carry

    lax.fori_loop(0, ROWS_PER_T, dchunk, 0)
    plsc.subcore_barrier()

    pltpu.sync_copy(deg_sh, dinvb)
    plsc.subcore_barrier()

    def newt(q, carry):
      sl = pl.ds(q * 16, 16)
      d = dinvb[sl]
      dinvb[sl] = _rsqrt16(d)
      return carry

    lax.fori_loop(0, NPAD // 16, newt, 0)

    pltpu.sync_copy(w2d.at[pl.ds(t * NROWS + wid * ROWS_PER_W, ROWS_PER_W)],
                    wown)

    def nchunk(k, carry):
      for j in range(CHUNK // 16):
        sl = pl.ds(j * 16, 16)
        dr = plsc.load_gather(dinvb, [rown[k, sl]])
        dc = plsc.load_gather(dinvb, [cown[k, sl]])
        nout[k, sl] = dr * wown[k, sl] * dc
      return carry

    lax.fori_loop(0, ROWS_PER_W, nchunk, 0)
    pltpu.sync_copy(
        nout, out.at[pl.ds(t * NROWS + wid * ROWS_PER_W, ROWS_PER_W)])


def _sc_norm(r2d, c2d, w2d):
  kfn = pl.kernel(
      _sc_norm_body,
      out_type=jax.ShapeDtypeStruct((SEQ * NROWS, CHUNK), jnp.float32),
      mesh=_sc_mesh(),
      compiler_params=_SC_PARAMS,
      scratch_types=[
          pltpu.VMEM((ROWS_PER_T, CHUNK), jnp.int32),
          pltpu.VMEM((ROWS_PER_T, CHUNK), jnp.float32),
          pltpu.VMEM((ROWS_PER_W, CHUNK), jnp.int32),
          pltpu.VMEM((ROWS_PER_W, CHUNK), jnp.int32),
          pltpu.VMEM((ROWS_PER_W, CHUNK), jnp.float32),
          pltpu.VMEM((NPAD,), jnp.float32),
          pltpu.VMEM((ROWS_PER_W, CHUNK), jnp.float32),
          pltpu.VMEM((STRIPE,), jnp.float32),
          pltpu.VMEM_SHARED((NPAD,), jnp.float32),
      ],
  )
  return kfn(r2d, c2d, w2d)


def _sc_msg_body(t0, t1, t2, r2d, c2d, nw2d, out,
                 rbuf, cbuf, nwbuf, normb, rb0, rb1, acc,
                 gs0, gs1, ss0, ss1):
  cid = lax.axis_index("c")
  sid = lax.axis_index("s")
  wid = cid * NS + sid
  tables = [t0, t1, t2]
  rowbufs = [rb0, rb1]
  gsems = [gs0, gs1]
  ssems = [ss0, ss1]

  pltpu.sync_copy(r2d.at[pl.ds(wid * ROWS_PER_W, ROWS_PER_W)], rbuf)
  pltpu.sync_copy(c2d.at[pl.ds(wid * ROWS_PER_W, ROWS_PER_W)], cbuf)

  def zrow_init(i, carry):
    for j in range(D // 16):
      rb0[i, pl.ds(j * 16, 16)] = jnp.zeros((16,), jnp.float32)
    return carry
  lax.fori_loop(0, CHUNK, zrow_init, 0)
  for z in range(STRIPE // CHUNK):
    pltpu.sync_copy(rb0, acc.at[pl.ds(sid * STRIPE + z * CHUNK, CHUNK)])
  plsc.subcore_barrier()

  nchunks = ROWS_PER_W
  nouter = nchunks // 2

  for t in range(SEQ):
    table = tables[t]
    pltpu.sync_copy(
        nw2d.at[pl.ds(t * NROWS + wid * ROWS_PER_W, ROWS_PER_W)], nwbuf)

    for b in range(2):
      pltpu.async_copy(table.at[rbuf.at[b]], rowbufs[b], gsems[b])

    def outer(m, carry):
      for b in range(2):
        k = 2 * m + b
        rowb = rowbufs[b]
        pltpu.make_async_copy(table.at[rbuf.at[k]], rowb, gsems[b]).wait()

        for j in range(CHUNK // 16):
          normb[pl.ds(j * 16, 16)] = nwbuf[k, pl.ds(j * 16, 16)]

        def scale(q, carry2):
          for u in range(4):
            e = q * 4 + u
            s = normb[pl.ds(e, 16)][0]
            for j in range(D // 16):
              sl = pl.ds(j * 16, 16)
              rowb[e, sl] = rowb[e, sl] * s
          return carry2
        lax.fori_loop(0, CHUNK // 4, scale, 0)

        pltpu.async_copy(rowb, acc.at[cbuf.at[k]], ssems[b], add=True)

        @pl.when(k < nchunks - 2)
        def _():
          pltpu.make_async_copy(rowb, acc.at[cbuf.at[k]], ssems[b]).wait()
          pltpu.async_copy(table.at[rbuf.at[k + 2]], rowb, gsems[b])
      return carry

    lax.fori_loop(0, nouter, outer, 0)

    for b in range(2):
      k = nchunks - 2 + b
      pltpu.make_async_copy(rowbufs[b], acc.at[cbuf.at[k]], ssems[b]).wait()
    plsc.subcore_barrier()

    pltpu.sync_copy(
        acc.at[pl.ds(sid * STRIPE, STRIPE)],
        out.at[cid * SEQ + t, pl.ds(sid * STRIPE, STRIPE)])
    if t < SEQ - 1:
      def zrow_again(i, carry):
        for j in range(D // 16):
          rb0[i, pl.ds(j * 16, 16)] = jnp.zeros((16,), jnp.float32)
        return carry
      lax.fori_loop(0, CHUNK, zrow_again, 0)
      for z in range(STRIPE // CHUNK):
        pltpu.sync_copy(rb0, acc.at[pl.ds(sid * STRIPE + z * CHUNK, CHUNK)])
      plsc.subcore_barrier()


def _sc_msg(t0, t1, t2, r2d, c2d, nw2d):
  kfn = pl.kernel(
      _sc_msg_body,
      out_type=jax.ShapeDtypeStruct((NC * SEQ, NPAD, D), jnp.float32),
      mesh=_sc_mesh(),
      compiler_params=_SC_PARAMS,
      scratch_types=[
          pltpu.VMEM((ROWS_PER_W, CHUNK), jnp.int32),
          pltpu.VMEM((ROWS_PER_W, CHUNK), jnp.int32),
          pltpu.VMEM((ROWS_PER_W, CHUNK), jnp.float32),
          pltpu.VMEM((CHUNK + 16,), jnp.float32),
          pltpu.VMEM((CHUNK, D), jnp.float32),
          pltpu.VMEM((CHUNK, D), jnp.float32),
          pltpu.VMEM_SHARED((NPAD, D), jnp.float32),
          pltpu.SemaphoreType.DMA,
          pltpu.SemaphoreType.DMA,
          pltpu.SemaphoreType.DMA,
          pltpu.SemaphoreType.DMA,
      ],
  )
  return kfn(t0, t1, t2, r2d, c2d, nw2d)


def _sc_lbuild_body(chol_hbm, out, slab, rowb):
  cid = lax.axis_index("c")
  sid = lax.axis_index("s")
  wid = cid * NS + sid
  iota = lax.broadcasted_iota(jnp.int32, (16,), 0)
  for m in range(CHOL_N // NW):
    i = wid * (CHOL_N // NW) + m
    off = (i * (i + 1)) // 2
    off_al = pl.multiple_of((off // 8) * 8, 8)
    sh = off - off_al
    pltpu.sync_copy(chol_hbm.at[pl.ds(off_al, CHOL_N + 24)], slab)
    for j in range(CHOL_N // 16):
      v = slab[pl.ds(sh + j * 16, 16)]
      pos = j * 16 + iota
      v = jnp.where(pos <= i, v, jnp.zeros((16,), jnp.float32))
      rowb[pl.ds(j * 16, 16)] = v
    pltpu.sync_copy(rowb, out.at[i])


def _sc_lbuild(chol_pad):
  kfn = pl.kernel(
      _sc_lbuild_body,
      out_type=jax.ShapeDtypeStruct((CHOL_N, CHOL_N), jnp.float32),
      mesh=_sc_mesh(),
      compiler_params=_SC_PARAMS,
      scratch_types=[
          pltpu.VMEM((CHOL_N + 24,), jnp.float32),
          pltpu.VMEM((CHOL_N,), jnp.float32),
      ],
  )
  return kfn(chol_pad)


_MM_BLK = 1200


def _tc_mm_body(x_ref, w_ref, o_ref):
  o_ref[...] = jnp.dot(x_ref[...], w_ref[...],
                       preferred_element_type=jnp.float32)


def _tc_mm(x2d, w):
  rows = x2d.shape[0]
  return pl.pallas_call(
      _tc_mm_body,
      grid=(rows // _MM_BLK,),
      in_specs=[
          pl.BlockSpec((_MM_BLK, D), lambda i: (i, 0)),
          pl.BlockSpec((D, D), lambda i: (0, 0)),
      ],
      out_specs=pl.BlockSpec((_MM_BLK, D), lambda i: (i, 0)),
      out_shape=jax.ShapeDtypeStruct((rows, D), jnp.float32),
  )(x2d, w)


_RB = 400


def _tc_fuse_mm_body(p0_ref, p1_ref, b_ref, w_ref, o_ref):
  h = jax.nn.relu(_nan2num(p0_ref[0] + p1_ref[0] + b_ref[...]))
  o_ref[0] = jnp.dot(h, w_ref[...], preferred_element_type=jnp.float32)


def _tc_fuse_mm(p0, p1, b, w):
  b2 = b.reshape(1, D)
  return pl.pallas_call(
      _tc_fuse_mm_body,
      grid=(SEQ, N // _RB),
      in_specs=[
          pl.BlockSpec((1, _RB, D), lambda t, i: (t, i, 0)),
          pl.BlockSpec((1, _RB, D), lambda t, i: (t, i, 0)),
          pl.BlockSpec((1, D), lambda t, i: (0, 0)),
          pl.BlockSpec((D, D), lambda t, i: (0, 0)),
      ],
      out_specs=pl.BlockSpec((1, _RB, D), lambda t, i: (t, i, 0)),
      out_shape=jax.ShapeDtypeStruct((SEQ, N, D), jnp.float32),
  )(p0, p1, b2, w)


def _tc_emb_body(p0_ref, p1_ref, b_ref, o_ref):
  t = pl.program_id(0)
  i = pl.program_id(1)
  h = jax.nn.relu(_nan2num(p0_ref[0] + p1_ref[0] + b_ref[...]))
  s = jnp.sum(h, axis=0, keepdims=True)
  row = pl.ds(t, 1)

  @pl.when(i == 0)
  def _():
    o_ref[row, :] = jnp.zeros((1, D), jnp.float32)

  o_ref[row, :] += s

  @pl.when(i == N // _RB - 1)
  def _():
    o_ref[row, :] = _nan2num(o_ref[row, :] / float(N))


def _tc_emb(p0, p1, b):
  b2 = b.reshape(1, D)
  return pl.pallas_call(
      _tc_emb_body,
      grid=(SEQ, N // _RB),
      in_specs=[
          pl.BlockSpec((1, _RB, D), lambda t, i: (t, i, 0)),
          pl.BlockSpec((1, _RB, D), lambda t, i: (t, i, 0)),
          pl.BlockSpec((1, D), lambda t, i: (0, 0)),
      ],
      out_specs=pl.BlockSpec((SEQ, D), lambda t, i: (0, 0)),
      out_shape=jax.ShapeDtypeStruct((SEQ, D), jnp.float32),
  )(p0, p1, b2)


def _tc_lstm_body(emb_ref, wih_ref, whh_ref, bih_ref, bhh_ref, o_ref):
  h = jnp.zeros((1, HID), jnp.float32)
  c = jnp.zeros((1, HID), jnp.float32)
  wih = wih_ref[...]
  whh = whh_ref[...]
  bias = bih_ref[...] + bhh_ref[...]
  dn = (((1,), (1,)), ((), ()))
  for t in range(SEQ):
    xt = emb_ref[pl.ds(t, 1), :]
    gates = (lax.dot_general(xt, wih, dn, preferred_element_type=jnp.float32)
             + lax.dot_general(h, whh, dn, preferred_element_type=jnp.float32)
             + bias)
    ig = jax.nn.sigmoid(gates[:, 0:HID])
    fg = jax.nn.sigmoid(gates[:, HID:2 * HID])
    gg = jnp.tanh(gates[:, 2 * HID:3 * HID])
    og = jax.nn.sigmoid(gates[:, 3 * HID:4 * HID])
    c = fg * c + ig * gg
    h = og * jnp.tanh(c)
  o_ref[...] = _nan2num(h)


def _tc_lstm(emb, wih, whh, bih, bhh):
  return pl.pallas_call(
      _tc_lstm_body,
      out_shape=jax.ShapeDtypeStruct((1, HID), jnp.float32),
  )(emb, wih, whh, bih.reshape(1, 4 * HID), bhh.reshape(1, 4 * HID))


_CB = 2304


def _tc_chol_body(fh_ref, w_ref, b_ref, o_ref):
  o_ref[...] = _nan2num(
      jnp.dot(fh_ref[...], w_ref[...], preferred_element_type=jnp.float32)
      + b_ref[...])


def _tc_chol(fh, w_fc, b_fc):
  return pl.pallas_call(
      _tc_chol_body,
      grid=(CHOL_ELEMS // _CB,),
      in_specs=[
          pl.BlockSpec((1, HID), lambda i: (0, 0)),
          pl.BlockSpec((HID, _CB), lambda i: (0, i)),
          pl.BlockSpec((1, _CB), lambda i: (0, i)),
      ],
      out_specs=pl.BlockSpec((1, _CB), lambda i: (0, i)),
      out_shape=jax.ShapeDtypeStruct((1, CHOL_ELEMS), jnp.float32),
  )(fh, w_fc, b_fc.reshape(1, CHOL_ELEMS))


def _tc_llt_body(l_ref, o_ref):
  l = l_ref[...]
  o_ref[...] = _nan2num(
      lax.dot_general(l, l, (((1,), (1,)), ((), ())),
                      preferred_element_type=jnp.float32))


def _tc_llt(l):
  return pl.pallas_call(
      _tc_llt_body,
      out_shape=jax.ShapeDtypeStruct((CHOL_N, CHOL_N), jnp.float32),
  )(l)


def kernel(x, edge_index, edge_weight, W1, b1, W2, b2,
           W_ih, W_hh, b_ih, b_hh, W_fc, b_fc):
  row, col = edge_index[0], edge_index[1]

  npad_e = E_PAD - E_EXT
  loop = jnp.arange(N, dtype=row.dtype)
  pad_r = jnp.arange(npad_e, dtype=row.dtype) % N
  pad_c = N + (jnp.arange(npad_e, dtype=row.dtype) % NDEAD)
  r_ext = jnp.concatenate([row, loop, pad_r])
  c_ext = jnp.concatenate([col, loop, pad_c])
  w_ext = jnp.concatenate(
      [edge_weight,
       jnp.ones((SEQ, N), jnp.float32),
       jnp.zeros((SEQ, npad_e), jnp.float32)], axis=1)

  r2d = r_ext.reshape(NROWS, CHUNK)
  c2d = c_ext.reshape(NROWS, CHUNK)
  w2d = w_ext.reshape(SEQ * NROWS, CHUNK)

  nw2d = _sc_norm(r2d, c2d, w2d)

  x2d = x.reshape(SEQ * N, D)
  xw1 = _tc_mm(x2d, W1)
  p1 = _sc_msg(xw1[0:N], xw1[N:2 * N], xw1[2 * N:3 * N], r2d, c2d, nw2d)
  p1 = p1.reshape(NC, SEQ, NPAD, D)

  xw2 = _tc_fuse_mm(p1[0], p1[1], b1, W2)
  p2 = _sc_msg(xw2[0], xw2[1], xw2[2], r2d, c2d, nw2d)
  p2 = p2.reshape(NC, SEQ, NPAD, D)

  emb = _tc_emb(p2[0], p2[1], b2)
  fh = _tc_lstm(emb, W_ih, W_hh, b_ih, b_hh)
  chol = _tc_chol(fh, W_fc, b_fc)

  chol_pad = jnp.concatenate(
      [chol.reshape(CHOL_ELEMS), jnp.zeros((CHOL_N,), jnp.float32)])
  l = _sc_lbuild(chol_pad)
  return _tc_llt(l)


# speedup vs baseline: 17.8665x; 1.1715x over previous
import jax
import jax.numpy as jnp
from jax import lax
from jax.experimental import pallas as pl
from jax.experimental.pallas import tpu as pltpu
from jax.experimental.pallas import tpu_sc as plsc

N = 10000
E = 320000
D = 128
HID = 128
SEQ = 3
CHOL_N = 512
CHOL_ELEMS = CHOL_N * (CHOL_N + 1) // 2
CLIP = 100000.0

NC = 2
NS = 16
NW = NC * NS

NPAD = 10240
NDEAD = 240
E_EXT = E + N
CHUNK = 64
ROWS_PER_W = 162
E_PAD = NW * ROWS_PER_W * CHUNK
NROWS = E_PAD // CHUNK
ROWS_PER_T = NROWS // NS
STRIPE = NPAD // NS

_SC_PARAMS = pltpu.CompilerParams(
    use_tc_tiling_on_sc=False, needs_layout_passes=False)


def _nan2num(t):
  return jnp.nan_to_num(t, nan=0.0, posinf=CLIP, neginf=-CLIP)


def _sc_mesh():
  return plsc.VectorSubcoreMesh(
      core_axis_name="c", subcore_axis_name="s",
      num_cores=NC, num_subcores=NS)


def _rsqrt16(d):
  half = d * 0.5
  ibits = plsc.bitcast(d, jnp.int32)
  y = plsc.bitcast(jnp.int32(0x5F3759DF) - lax.shift_right_logical(ibits, 1),
                   jnp.float32)
  for _ in range(3):
    y = y * (1.5 - half * y * y)
  return y


def _sc_norm_body(r2d, c2d, w2d, out,
                  call_buf, wall_buf, rown, cown, wown, dinvb, nout, zbuf,
                  deg_sh, dsem):
  cid = lax.axis_index("c")
  sid = lax.axis_index("s")
  wid = cid * NS + sid

  pltpu.sync_copy(c2d.at[pl.ds(sid * ROWS_PER_T, ROWS_PER_T)], call_buf)
  pltpu.sync_copy(r2d.at[pl.ds(wid * ROWS_PER_W, ROWS_PER_W)], rown)
  pltpu.sync_copy(c2d.at[pl.ds(wid * ROWS_PER_W, ROWS_PER_W)], cown)

  for i in range(STRIPE // 16):
    zbuf[pl.ds(i * 16, 16)] = jnp.zeros((16,), jnp.float32)

  for t in range(SEQ):
    pltpu.sync_copy(zbuf, deg_sh.at[pl.ds(sid * STRIPE, STRIPE)])
    plsc.subcore_barrier()

    pltpu.sync_copy(w2d.at[pl.ds(t * NROWS + sid * ROWS_PER_T, ROWS_PER_T)],
                    wall_buf)
    grp = 6
    ngrp = ROWS_PER_T // grp
    for u in range(grp):
      pltpu.async_copy(wall_buf.at[u], deg_sh.at[call_buf.at[u]], dsem,
                       add=True)

    def dgroup(g, carry):
      @pl.when(g < ngrp - 1)
      def _():
        for u in range(grp):
          k = (g + 1) * grp + u
          pltpu.async_copy(wall_buf.at[k], deg_sh.at[call_buf.at[k]], dsem,
                           add=True)
      for u in range(grp):
        k = g * grp + u
        pltpu.make_async_copy(wall_buf.at[k], deg_sh.at[call_buf.at[k]],
                              dsem).wait()
      return carry

    lax.fori_loop(0, ngrp, dgroup, 0)
    plsc.subcore_barrier()

    pltpu.sync_copy(deg_sh, dinvb)
    plsc.subcore_barrier()

    @plsc.parallel_loop(0, NPAD // 16, unroll=4)
    def _(q):
      sl = pl.ds(q * 16, 16)
      dinvb[sl] = _rsqrt16(dinvb[sl])

    pltpu.sync_copy(w2d.at[pl.ds(t * NROWS + wid * ROWS_PER_W, ROWS_PER_W)],
                    wown)

    @plsc.parallel_loop(0, ROWS_PER_W, unroll=2)
    def _(k):
      for j in range(CHUNK // 16):
        sl = pl.ds(j * 16, 16)
        dr = plsc.load_gather(dinvb, [rown[k, sl]])
        dc = plsc.load_gather(dinvb, [cown[k, sl]])
        nout[k, sl] = dr * wown[k, sl] * dc
    pltpu.sync_copy(
        nout, out.at[pl.ds(t * NROWS + wid * ROWS_PER_W, ROWS_PER_W)])


def _sc_norm(r2d, c2d, w2d):
  kfn = pl.kernel(
      _sc_norm_body,
      out_type=jax.ShapeDtypeStruct((SEQ * NROWS, CHUNK), jnp.float32),
      mesh=_sc_mesh(),
      compiler_params=_SC_PARAMS,
      scratch_types=[
          pltpu.VMEM((ROWS_PER_T, CHUNK), jnp.int32),
          pltpu.VMEM((ROWS_PER_T, CHUNK), jnp.float32),
          pltpu.VMEM((ROWS_PER_W, CHUNK), jnp.int32),
          pltpu.VMEM((ROWS_PER_W, CHUNK), jnp.int32),
          pltpu.VMEM((ROWS_PER_W, CHUNK), jnp.float32),
          pltpu.VMEM((NPAD,), jnp.float32),
          pltpu.VMEM((ROWS_PER_W, CHUNK), jnp.float32),
          pltpu.VMEM((STRIPE,), jnp.float32),
          pltpu.VMEM_SHARED((NPAD,), jnp.float32),
          pltpu.SemaphoreType.DMA,
      ],
  )
  return kfn(r2d, c2d, w2d)


def _sc_msg_body(t0, t1, t2, r2d, c2d, nw2d, out,
                 rbuf, cbuf, nwbuf, normb, rb0, rb1, acc,
                 gs0, gs1, ss0, ss1):
  cid = lax.axis_index("c")
  sid = lax.axis_index("s")
  wid = cid * NS + sid
  tables = [t0, t1, t2]
  rowbufs = [rb0, rb1]
  gsems = [gs0, gs1]
  ssems = [ss0, ss1]

  pltpu.sync_copy(r2d.at[pl.ds(wid * ROWS_PER_W, ROWS_PER_W)], rbuf)
  pltpu.sync_copy(c2d.at[pl.ds(wid * ROWS_PER_W, ROWS_PER_W)], cbuf)

  def zrow_init(i, carry):
    for j in range(D // 16):
      rb0[i, pl.ds(j * 16, 16)] = jnp.zeros((16,), jnp.float32)
    return carry
  lax.fori_loop(0, CHUNK, zrow_init, 0)
  for z in range(STRIPE // CHUNK):
    pltpu.sync_copy(rb0, acc.at[pl.ds(sid * STRIPE + z * CHUNK, CHUNK)])
  plsc.subcore_barrier()

  nchunks = ROWS_PER_W
  nouter = nchunks // 2

  for t in range(SEQ):
    table = tables[t]
    pltpu.sync_copy(
        nw2d.at[pl.ds(t * NROWS + wid * ROWS_PER_W, ROWS_PER_W)], nwbuf)

    for b in range(2):
      pltpu.async_copy(table.at[rbuf.at[b]], rowbufs[b], gsems[b])

    def outer(m, carry):
      for b in range(2):
        k = 2 * m + b
        rowb = rowbufs[b]
        pltpu.make_async_copy(table.at[rbuf.at[k]], rowb, gsems[b]).wait()

        for j in range(CHUNK // 16):
          normb[pl.ds(j * 16, 16)] = nwbuf[k, pl.ds(j * 16, 16)]

        @plsc.parallel_loop(0, CHUNK, unroll=4)
        def _(e):
          s = normb[pl.ds(e, 16)][0]
          for j in range(D // 16):
            sl = pl.ds(j * 16, 16)
            rowb[e, sl] = rowb[e, sl] * s

        pltpu.async_copy(rowb, acc.at[cbuf.at[k]], ssems[b], add=True)

        @pl.when(k < nchunks - 2)
        def _():
          pltpu.make_async_copy(rowb, acc.at[cbuf.at[k]], ssems[b]).wait()
          pltpu.async_copy(table.at[rbuf.at[k + 2]], rowb, gsems[b])
      return carry

    lax.fori_loop(0, nouter, outer, 0)

    for b in range(2):
      k = nchunks - 2 + b
      pltpu.make_async_copy(rowbufs[b], acc.at[cbuf.at[k]], ssems[b]).wait()
    plsc.subcore_barrier()

    pltpu.sync_copy(
        acc.at[pl.ds(sid * STRIPE, STRIPE)],
        out.at[cid * SEQ + t, pl.ds(sid * STRIPE, STRIPE)])
    if t < SEQ - 1:
      def zrow_again(i, carry):
        for j in range(D // 16):
          rb0[i, pl.ds(j * 16, 16)] = jnp.zeros((16,), jnp.float32)
        return carry
      lax.fori_loop(0, CHUNK, zrow_again, 0)
      for z in range(STRIPE // CHUNK):
        pltpu.sync_copy(rb0, acc.at[pl.ds(sid * STRIPE + z * CHUNK, CHUNK)])
      plsc.subcore_barrier()


def _sc_msg(t0, t1, t2, r2d, c2d, nw2d):
  kfn = pl.kernel(
      _sc_msg_body,
      out_type=jax.ShapeDtypeStruct((NC * SEQ, NPAD, D), jnp.float32),
      mesh=_sc_mesh(),
      compiler_params=_SC_PARAMS,
      scratch_types=[
          pltpu.VMEM((ROWS_PER_W, CHUNK), jnp.int32),
          pltpu.VMEM((ROWS_PER_W, CHUNK), jnp.int32),
          pltpu.VMEM((ROWS_PER_W, CHUNK), jnp.float32),
          pltpu.VMEM((CHUNK + 16,), jnp.float32),
          pltpu.VMEM((CHUNK, D), jnp.float32),
          pltpu.VMEM((CHUNK, D), jnp.float32),
          pltpu.VMEM_SHARED((NPAD, D), jnp.float32),
          pltpu.SemaphoreType.DMA,
          pltpu.SemaphoreType.DMA,
          pltpu.SemaphoreType.DMA,
          pltpu.SemaphoreType.DMA,
      ],
  )
  return kfn(t0, t1, t2, r2d, c2d, nw2d)


def _sc_lbuild_body(chol_hbm, out, slab, rowb):
  cid = lax.axis_index("c")
  sid = lax.axis_index("s")
  wid = cid * NS + sid
  iota = lax.broadcasted_iota(jnp.int32, (16,), 0)
  for m in range(CHOL_N // NW):
    i = wid * (CHOL_N // NW) + m
    off = (i * (i + 1)) // 2
    off_al = pl.multiple_of((off // 8) * 8, 8)
    sh = off - off_al
    pltpu.sync_copy(chol_hbm.at[pl.ds(off_al, CHOL_N + 24)], slab)
    for j in range(CHOL_N // 16):
      v = slab[pl.ds(sh + j * 16, 16)]
      pos = j * 16 + iota
      v = jnp.where(pos <= i, v, jnp.zeros((16,), jnp.float32))
      rowb[pl.ds(j * 16, 16)] = v
    pltpu.sync_copy(rowb, out.at[i])


def _sc_lbuild(chol_pad):
  kfn = pl.kernel(
      _sc_lbuild_body,
      out_type=jax.ShapeDtypeStruct((CHOL_N, CHOL_N), jnp.float32),
      mesh=_sc_mesh(),
      compiler_params=_SC_PARAMS,
      scratch_types=[
          pltpu.VMEM((CHOL_N + 24,), jnp.float32),
          pltpu.VMEM((CHOL_N,), jnp.float32),
      ],
  )
  return kfn(chol_pad)


_MM_BLK = 1200


def _tc_mm_body(x_ref, w_ref, o_ref):
  o_ref[...] = jnp.dot(x_ref[...], w_ref[...],
                       preferred_element_type=jnp.float32)


def _tc_mm(x2d, w):
  rows = x2d.shape[0]
  return pl.pallas_call(
      _tc_mm_body,
      grid=(rows // _MM_BLK,),
      in_specs=[
          pl.BlockSpec((_MM_BLK, D), lambda i: (i, 0)),
          pl.BlockSpec((D, D), lambda i: (0, 0)),
      ],
      out_specs=pl.BlockSpec((_MM_BLK, D), lambda i: (i, 0)),
      out_shape=jax.ShapeDtypeStruct((rows, D), jnp.float32),
  )(x2d, w)


_RB = 400


def _tc_fuse_mm_body(p0_ref, p1_ref, b_ref, w_ref, o_ref):
  h = jax.nn.relu(_nan2num(p0_ref[0] + p1_ref[0] + b_ref[...]))
  o_ref[0] = jnp.dot(h, w_ref[...], preferred_element_type=jnp.float32)


def _tc_fuse_mm(p0, p1, b, w):
  b2 = b.reshape(1, D)
  return pl.pallas_call(
      _tc_fuse_mm_body,
      grid=(SEQ, N // _RB),
      in_specs=[
          pl.BlockSpec((1, _RB, D), lambda t, i: (t, i, 0)),
          pl.BlockSpec((1, _RB, D), lambda t, i: (t, i, 0)),
          pl.BlockSpec((1, D), lambda t, i: (0, 0)),
          pl.BlockSpec((D, D), lambda t, i: (0, 0)),
      ],
      out_specs=pl.BlockSpec((1, _RB, D), lambda t, i: (t, i, 0)),
      out_shape=jax.ShapeDtypeStruct((SEQ, N, D), jnp.float32),
  )(p0, p1, b2, w)


def _tc_emb_body(p0_ref, p1_ref, b_ref, o_ref):
  t = pl.program_id(0)
  i = pl.program_id(1)
  h = jax.nn.relu(_nan2num(p0_ref[0] + p1_ref[0] + b_ref[...]))
  s = jnp.sum(h, axis=0, keepdims=True)
  row = pl.ds(t, 1)

  @pl.when(i == 0)
  def _():
    o_ref[row, :] = jnp.zeros((1, D), jnp.float32)

  o_ref[row, :] += s

  @pl.when(i == N // _RB - 1)
  def _():
    o_ref[row, :] = _nan2num(o_ref[row, :] / float(N))


def _tc_emb(p0, p1, b):
  b2 = b.reshape(1, D)
  return pl.pallas_call(
      _tc_emb_body,
      grid=(SEQ, N // _RB),
      in_specs=[
          pl.BlockSpec((1, _RB, D), lambda t, i: (t, i, 0)),
          pl.BlockSpec((1, _RB, D), lambda t, i: (t, i, 0)),
          pl.BlockSpec((1, D), lambda t, i: (0, 0)),
      ],
      out_specs=pl.BlockSpec((SEQ, D), lambda t, i: (0, 0)),
      out_shape=jax.ShapeDtypeStruct((SEQ, D), jnp.float32),
  )(p0, p1, b2)


def _tc_lstm_body(emb_ref, wih_ref, whh_ref, bih_ref, bhh_ref, o_ref):
  h = jnp.zeros((1, HID), jnp.float32)
  c = jnp.zeros((1, HID), jnp.float32)
  wih = wih_ref[...]
  whh = whh_ref[...]
  bias = bih_ref[...] + bhh_ref[...]
  dn = (((1,), (1,)), ((), ()))
  for t in range(SEQ):
    xt = emb_ref[pl.ds(t, 1), :]
    gates = (lax.dot_general(xt, wih, dn, preferred_element_type=jnp.float32)
             + lax.dot_general(h, whh, dn, preferred_element_type=jnp.float32)
             + bias)
    ig = jax.nn.sigmoid(gates[:, 0:HID])
    fg = jax.nn.sigmoid(gates[:, HID:2 * HID])
    gg = jnp.tanh(gates[:, 2 * HID:3 * HID])
    og = jax.nn.sigmoid(gates[:, 3 * HID:4 * HID])
    c = fg * c + ig * gg
    h = og * jnp.tanh(c)
  o_ref[...] = _nan2num(h)


def _tc_lstm(emb, wih, whh, bih, bhh):
  return pl.pallas_call(
      _tc_lstm_body,
      out_shape=jax.ShapeDtypeStruct((1, HID), jnp.float32),
  )(emb, wih, whh, bih.reshape(1, 4 * HID), bhh.reshape(1, 4 * HID))


_CB = 2304


def _tc_chol_body(fh_ref, w_ref, b_ref, o_ref):
  o_ref[...] = _nan2num(
      jnp.dot(fh_ref[...], w_ref[...], preferred_element_type=jnp.float32)
      + b_ref[...])


def _tc_chol(fh, w_fc, b_fc):
  return pl.pallas_call(
      _tc_chol_body,
      grid=(CHOL_ELEMS // _CB,),
      in_specs=[
          pl.BlockSpec((1, HID), lambda i: (0, 0)),
          pl.BlockSpec((HID, _CB), lambda i: (0, i)),
          pl.BlockSpec((1, _CB), lambda i: (0, i)),
      ],
      out_specs=pl.BlockSpec((1, _CB), lambda i: (0, i)),
      out_shape=jax.ShapeDtypeStruct((1, CHOL_ELEMS), jnp.float32),
  )(fh, w_fc, b_fc.reshape(1, CHOL_ELEMS))


def _tc_llt_body(l_ref, o_ref):
  l = l_ref[...]
  o_ref[...] = _nan2num(
      lax.dot_general(l, l, (((1,), (1,)), ((), ())),
                      preferred_element_type=jnp.float32))


def _tc_llt(l):
  return pl.pallas_call(
      _tc_llt_body,
      out_shape=jax.ShapeDtypeStruct((CHOL_N, CHOL_N), jnp.float32),
  )(l)


def kernel(x, edge_index, edge_weight, W1, b1, W2, b2,
           W_ih, W_hh, b_ih, b_hh, W_fc, b_fc):
  row, col = edge_index[0], edge_index[1]

  npad_e = E_PAD - E_EXT
  loop = jnp.arange(N, dtype=row.dtype)
  pad_r = jnp.arange(npad_e, dtype=row.dtype) % N
  pad_c = N + (jnp.arange(npad_e, dtype=row.dtype) % NDEAD)
  r_ext = jnp.concatenate([row, loop, pad_r])
  c_ext = jnp.concatenate([col, loop, pad_c])
  w_ext = jnp.concatenate(
      [edge_weight,
       jnp.ones((SEQ, N), jnp.float32),
       jnp.zeros((SEQ, npad_e), jnp.float32)], axis=1)

  r2d = r_ext.reshape(NROWS, CHUNK)
  c2d = c_ext.reshape(NROWS, CHUNK)
  w2d = w_ext.reshape(SEQ * NROWS, CHUNK)

  nw2d = _sc_norm(r2d, c2d, w2d)

  x2d = x.reshape(SEQ * N, D)
  xw1 = _tc_mm(x2d, W1)
  p1 = _sc_msg(xw1[0:N], xw1[N:2 * N], xw1[2 * N:3 * N], r2d, c2d, nw2d)
  p1 = p1.reshape(NC, SEQ, NPAD, D)

  xw2 = _tc_fuse_mm(p1[0], p1[1], b1, W2)
  p2 = _sc_msg(xw2[0], xw2[1], xw2[2], r2d, c2d, nw2d)
  p2 = p2.reshape(NC, SEQ, NPAD, D)

  emb = _tc_emb(p2[0], p2[1], b2)
  fh = _tc_lstm(emb, W_ih, W_hh, b_ih, b_hh)
  chol = _tc_chol(fh, W_fc, b_fc)

  chol_pad = jnp.concatenate(
      [chol.reshape(CHOL_ELEMS), jnp.zeros((CHOL_N,), jnp.float32)])
  l = _sc_lbuild(chol_pad)
  return _tc_llt(l)


# speedup vs baseline: 20.2366x; 1.1327x over previous
import jax
import jax.numpy as jnp
from jax import lax
from jax.experimental import pallas as pl
from jax.experimental.pallas import tpu as pltpu
from jax.experimental.pallas import tpu_sc as plsc

N = 10000
E = 320000
D = 128
HID = 128
SEQ = 3
CHOL_N = 512
CHOL_ELEMS = CHOL_N * (CHOL_N + 1) // 2
CLIP = 100000.0

NC = 2
NS = 16
NW = NC * NS

NPAD = 10240
NDEAD = 240
E_EXT = E + N
CHUNK = 64
ROWS_PER_W = 162
E_PAD = NW * ROWS_PER_W * CHUNK
NROWS = E_PAD // CHUNK
ROWS_PER_T = NROWS // NS
STRIPE = NPAD // NS

_SC_PARAMS = pltpu.CompilerParams(
    use_tc_tiling_on_sc=False, needs_layout_passes=False)


def _nan2num(t):
  return jnp.nan_to_num(t, nan=0.0, posinf=CLIP, neginf=-CLIP)


def _sc_mesh():
  return plsc.VectorSubcoreMesh(
      core_axis_name="c", subcore_axis_name="s",
      num_cores=NC, num_subcores=NS)


def _rsqrt16(d):
  half = d * 0.5
  ibits = plsc.bitcast(d, jnp.int32)
  y = plsc.bitcast(jnp.int32(0x5F3759DF) - lax.shift_right_logical(ibits, 1),
                   jnp.float32)
  for _ in range(3):
    y = y * (1.5 - half * y * y)
  return y


def _sc_norm_body(r2d, c2d, w2d, out,
                  call_buf, wall_buf, rown, cown, wown, dinvb, nout, zbuf,
                  deg_sh, dsem):
  cid = lax.axis_index("c")
  sid = lax.axis_index("s")
  wid = cid * NS + sid

  pltpu.sync_copy(c2d.at[pl.ds(sid * ROWS_PER_T, ROWS_PER_T)], call_buf)
  pltpu.sync_copy(r2d.at[pl.ds(wid * ROWS_PER_W, ROWS_PER_W)], rown)
  pltpu.sync_copy(c2d.at[pl.ds(wid * ROWS_PER_W, ROWS_PER_W)], cown)

  for i in range(STRIPE // 16):
    zbuf[pl.ds(i * 16, 16)] = jnp.zeros((16,), jnp.float32)

  for t in range(SEQ):
    pltpu.sync_copy(zbuf, deg_sh.at[pl.ds(sid * STRIPE, STRIPE)])
    plsc.subcore_barrier()

    pltpu.sync_copy(w2d.at[pl.ds(t * NROWS + sid * ROWS_PER_T, ROWS_PER_T)],
                    wall_buf)
    grp = 6
    ngrp = ROWS_PER_T // grp
    for u in range(grp):
      pltpu.async_copy(wall_buf.at[u], deg_sh.at[call_buf.at[u]], dsem,
                       add=True)

    def dgroup(g, carry):
      @pl.when(g < ngrp - 1)
      def _():
        for u in range(grp):
          k = (g + 1) * grp + u
          pltpu.async_copy(wall_buf.at[k], deg_sh.at[call_buf.at[k]], dsem,
                           add=True)
      for u in range(grp):
        k = g * grp + u
        pltpu.make_async_copy(wall_buf.at[k], deg_sh.at[call_buf.at[k]],
                              dsem).wait()
      return carry

    lax.fori_loop(0, ngrp, dgroup, 0)
    plsc.subcore_barrier()

    pltpu.sync_copy(deg_sh, dinvb)
    plsc.subcore_barrier()

    @plsc.parallel_loop(0, NPAD // 16, unroll=4)
    def _(q):
      sl = pl.ds(q * 16, 16)
      dinvb[sl] = _rsqrt16(dinvb[sl])

    pltpu.sync_copy(w2d.at[pl.ds(t * NROWS + wid * ROWS_PER_W, ROWS_PER_W)],
                    wown)

    @plsc.parallel_loop(0, ROWS_PER_W, unroll=2)
    def _(k):
      for j in range(CHUNK // 16):
        sl = pl.ds(j * 16, 16)
        dr = plsc.load_gather(dinvb, [rown[k, sl]])
        dc = plsc.load_gather(dinvb, [cown[k, sl]])
        nout[k, sl] = dr * wown[k, sl] * dc
    pltpu.sync_copy(
        nout, out.at[pl.ds(t * NROWS + wid * ROWS_PER_W, ROWS_PER_W)])


def _sc_norm(r2d, c2d, w2d):
  kfn = pl.kernel(
      _sc_norm_body,
      out_type=jax.ShapeDtypeStruct((SEQ * NROWS, CHUNK), jnp.float32),
      mesh=_sc_mesh(),
      compiler_params=_SC_PARAMS,
      scratch_types=[
          pltpu.VMEM((ROWS_PER_T, CHUNK), jnp.int32),
          pltpu.VMEM((ROWS_PER_T, CHUNK), jnp.float32),
          pltpu.VMEM((ROWS_PER_W, CHUNK), jnp.int32),
          pltpu.VMEM((ROWS_PER_W, CHUNK), jnp.int32),
          pltpu.VMEM((ROWS_PER_W, CHUNK), jnp.float32),
          pltpu.VMEM((NPAD,), jnp.float32),
          pltpu.VMEM((ROWS_PER_W, CHUNK), jnp.float32),
          pltpu.VMEM((STRIPE,), jnp.float32),
          pltpu.VMEM_SHARED((NPAD,), jnp.float32),
          pltpu.SemaphoreType.DMA,
      ],
  )
  return kfn(r2d, c2d, w2d)


def _sc_msg_body(t0, t1, t2, r2d, c2d, nw2d, out,
                 rbuf, cbuf, rb0, rb1, rb2, nr0, nr1, nr2, acc,
                 gs0, gs1, gs2, ss0, ss1, ss2):
  cid = lax.axis_index("c")
  sid = lax.axis_index("s")
  wid = cid * NS + sid
  tables = [t0, t1, t2]
  rowbufs = [rb0, rb1, rb2]
  nrings = [nr0, nr1, nr2]
  gsems = [gs0, gs1, gs2]
  ssems = [ss0, ss1, ss2]
  nbase = wid * ROWS_PER_W

  pltpu.sync_copy(r2d.at[pl.ds(nbase, ROWS_PER_W)], rbuf)
  pltpu.sync_copy(c2d.at[pl.ds(nbase, ROWS_PER_W)], cbuf)

  def zrow_init(i, carry):
    for j in range(D // 16):
      rb0[i, pl.ds(j * 16, 16)] = jnp.zeros((16,), jnp.float32)
    return carry
  lax.fori_loop(0, CHUNK, zrow_init, 0)
  for z in range(STRIPE // CHUNK):
    pltpu.sync_copy(rb0, acc.at[pl.ds(sid * STRIPE + z * CHUNK, CHUNK)])
  plsc.subcore_barrier()

  nchunks = ROWS_PER_W
  nouter = nchunks // 3

  for t in range(SEQ):
    table = tables[t]
    trow = t * NROWS + nbase

    def issue(k, b):
      pltpu.async_copy(table.at[rbuf.at[k]], rowbufs[b], gsems[b])
      pltpu.async_copy(nw2d.at[trow + k], nrings[b].at[pl.ds(0, CHUNK)],
                       gsems[b])

    def wait_in(k, b):
      pltpu.make_async_copy(table.at[rbuf.at[k]], rowbufs[b], gsems[b]).wait()
      pltpu.make_async_copy(nw2d.at[trow + k], nrings[b].at[pl.ds(0, CHUNK)],
                            gsems[b]).wait()

    def wait_sc(k, b):
      pltpu.make_async_copy(rowbufs[b], acc.at[cbuf.at[k]], ssems[b]).wait()

    issue(0, 0)
    issue(1, 1)

    def outer(m, carry):
      for b in range(3):
        k = 3 * m + b
        rowb = rowbufs[b]
        nring = nrings[b]
        wait_in(k, b)

        @plsc.parallel_loop(0, CHUNK, unroll=4)
        def _(e):
          s = nring[pl.ds(e, 16)][0]
          for j in range(D // 16):
            sl = pl.ds(j * 16, 16)
            rowb[e, sl] = rowb[e, sl] * s

        pltpu.async_copy(rowb, acc.at[cbuf.at[k]], ssems[b], add=True)

        bp = (b + 2) % 3
        if b == 0:
          @pl.when(m > 0)
          def _():
            wait_sc(k - 1, bp)
            issue(k + 2, bp)

          @pl.when(m == 0)
          def _():
            issue(k + 2, bp)
        else:
          @pl.when(m < nouter - 1)
          def _():
            wait_sc(k - 1, bp)
            issue(k + 2, bp)

          @pl.when(m == nouter - 1)
          def _():
            wait_sc(k - 1, bp)
      return carry

    lax.fori_loop(0, nouter, outer, 0)

    wait_sc(nchunks - 1, (nchunks - 1) % 3)
    plsc.subcore_barrier()

    pltpu.sync_copy(
        acc.at[pl.ds(sid * STRIPE, STRIPE)],
        out.at[cid * SEQ + t, pl.ds(sid * STRIPE, STRIPE)])
    if t < SEQ - 1:
      def zrow_again(i, carry):
        for j in range(D // 16):
          rb0[i, pl.ds(j * 16, 16)] = jnp.zeros((16,), jnp.float32)
        return carry
      lax.fori_loop(0, CHUNK, zrow_again, 0)
      for z in range(STRIPE // CHUNK):
        pltpu.sync_copy(rb0, acc.at[pl.ds(sid * STRIPE + z * CHUNK, CHUNK)])
      plsc.subcore_barrier()


def _sc_msg(t0, t1, t2, r2d, c2d, nw2d):
  kfn = pl.kernel(
      _sc_msg_body,
      out_type=jax.ShapeDtypeStruct((NC * SEQ, NPAD, D), jnp.float32),
      mesh=_sc_mesh(),
      compiler_params=_SC_PARAMS,
      scratch_types=[
          pltpu.VMEM((ROWS_PER_W, CHUNK), jnp.int32),
          pltpu.VMEM((ROWS_PER_W, CHUNK), jnp.int32),
          pltpu.VMEM((CHUNK, D), jnp.float32),
          pltpu.VMEM((CHUNK, D), jnp.float32),
          pltpu.VMEM((CHUNK, D), jnp.float32),
          pltpu.VMEM((CHUNK + 16,), jnp.float32),
          pltpu.VMEM((CHUNK + 16,), jnp.float32),
          pltpu.VMEM((CHUNK + 16,), jnp.float32),
          pltpu.VMEM_SHARED((NPAD, D), jnp.float32),
          pltpu.SemaphoreType.DMA,
          pltpu.SemaphoreType.DMA,
          pltpu.SemaphoreType.DMA,
          pltpu.SemaphoreType.DMA,
          pltpu.SemaphoreType.DMA,
          pltpu.SemaphoreType.DMA,
      ],
  )
  return kfn(t0, t1, t2, r2d, c2d, nw2d)


def _sc_lbuild_body(chol_hbm, out, slab, rowb):
  cid = lax.axis_index("c")
  sid = lax.axis_index("s")
  wid = cid * NS + sid
  iota = lax.broadcasted_iota(jnp.int32, (16,), 0)
  for m in range(CHOL_N // NW):
    i = wid * (CHOL_N // NW) + m
    off = (i * (i + 1)) // 2
    off_al = pl.multiple_of((off // 8) * 8, 8)
    sh = off - off_al
    pltpu.sync_copy(chol_hbm.at[pl.ds(off_al, CHOL_N + 24)], slab)
    for j in range(CHOL_N // 16):
      v = slab[pl.ds(sh + j * 16, 16)]
      pos = j * 16 + iota
      v = jnp.where(pos <= i, v, jnp.zeros((16,), jnp.float32))
      rowb[pl.ds(j * 16, 16)] = v
    pltpu.sync_copy(rowb, out.at[i])


def _sc_lbuild(chol_pad):
  kfn = pl.kernel(
      _sc_lbuild_body,
      out_type=jax.ShapeDtypeStruct((CHOL_N, CHOL_N), jnp.float32),
      mesh=_sc_mesh(),
      compiler_params=_SC_PARAMS,
      scratch_types=[
          pltpu.VMEM((CHOL_N + 24,), jnp.float32),
          pltpu.VMEM((CHOL_N,), jnp.float32),
      ],
  )
  return kfn(chol_pad)


_MM_BLK = 1200


def _tc_mm_body(x_ref, w_ref, o_ref):
  o_ref[...] = jnp.dot(x_ref[...], w_ref[...],
                       preferred_element_type=jnp.float32)


def _tc_mm(x2d, w):
  rows = x2d.shape[0]
  return pl.pallas_call(
      _tc_mm_body,
      grid=(rows // _MM_BLK,),
      in_specs=[
          pl.BlockSpec((_MM_BLK, D), lambda i: (i, 0)),
          pl.BlockSpec((D, D), lambda i: (0, 0)),
      ],
      out_specs=pl.BlockSpec((_MM_BLK, D), lambda i: (i, 0)),
      out_shape=jax.ShapeDtypeStruct((rows, D), jnp.float32),
  )(x2d, w)


_RB = 400


def _tc_fuse_mm_body(p0_ref, p1_ref, b_ref, w_ref, o_ref):
  h = jax.nn.relu(_nan2num(p0_ref[0] + p1_ref[0] + b_ref[...]))
  o_ref[0] = jnp.dot(h, w_ref[...], preferred_element_type=jnp.float32)


def _tc_fuse_mm(p0, p1, b, w):
  b2 = b.reshape(1, D)
  return pl.pallas_call(
      _tc_fuse_mm_body,
      grid=(SEQ, N // _RB),
      in_specs=[
          pl.BlockSpec((1, _RB, D), lambda t, i: (t, i, 0)),
          pl.BlockSpec((1, _RB, D), lambda t, i: (t, i, 0)),
          pl.BlockSpec((1, D), lambda t, i: (0, 0)),
          pl.BlockSpec((D, D), lambda t, i: (0, 0)),
      ],
      out_specs=pl.BlockSpec((1, _RB, D), lambda t, i: (t, i, 0)),
      out_shape=jax.ShapeDtypeStruct((SEQ, N, D), jnp.float32),
  )(p0, p1, b2, w)


def _tc_emb_body(p0_ref, p1_ref, b_ref, o_ref):
  t = pl.program_id(0)
  i = pl.program_id(1)
  h = jax.nn.relu(_nan2num(p0_ref[0] + p1_ref[0] + b_ref[...]))
  s = jnp.sum(h, axis=0, keepdims=True)
  row = pl.ds(t, 1)

  @pl.when(i == 0)
  def _():
    o_ref[row, :] = jnp.zeros((1, D), jnp.float32)

  o_ref[row, :] += s

  @pl.when(i == N // _RB - 1)
  def _():
    o_ref[row, :] = _nan2num(o_ref[row, :] / float(N))


def _tc_emb(p0, p1, b):
  b2 = b.reshape(1, D)
  return pl.pallas_call(
      _tc_emb_body,
      grid=(SEQ, N // _RB),
      in_specs=[
          pl.BlockSpec((1, _RB, D), lambda t, i: (t, i, 0)),
          pl.BlockSpec((1, _RB, D), lambda t, i: (t, i, 0)),
          pl.BlockSpec((1, D), lambda t, i: (0, 0)),
      ],
      out_specs=pl.BlockSpec((SEQ, D), lambda t, i: (0, 0)),
      out_shape=jax.ShapeDtypeStruct((SEQ, D), jnp.float32),
  )(p0, p1, b2)


def _tc_lstm_body(emb_ref, wih_ref, whh_ref, bih_ref, bhh_ref, o_ref):
  h = jnp.zeros((1, HID), jnp.float32)
  c = jnp.zeros((1, HID), jnp.float32)
  wih = wih_ref[...]
  whh = whh_ref[...]
  bias = bih_ref[...] + bhh_ref[...]
  dn = (((1,), (1,)), ((), ()))
  for t in range(SEQ):
    xt = emb_ref[pl.ds(t, 1), :]
    gates = (lax.dot_general(xt, wih, dn, preferred_element_type=jnp.float32)
             + lax.dot_general(h, whh, dn, preferred_element_type=jnp.float32)
             + bias)
    ig = jax.nn.sigmoid(gates[:, 0:HID])
    fg = jax.nn.sigmoid(gates[:, HID:2 * HID])
    gg = jnp.tanh(gates[:, 2 * HID:3 * HID])
    og = jax.nn.sigmoid(gates[:, 3 * HID:4 * HID])
    c = fg * c + ig * gg
    h = og * jnp.tanh(c)
  o_ref[...] = _nan2num(h)


def _tc_lstm(emb, wih, whh, bih, bhh):
  return pl.pallas_call(
      _tc_lstm_body,
      out_shape=jax.ShapeDtypeStruct((1, HID), jnp.float32),
  )(emb, wih, whh, bih.reshape(1, 4 * HID), bhh.reshape(1, 4 * HID))


_CB = 2304


def _tc_chol_body(fh_ref, w_ref, b_ref, o_ref):
  o_ref[...] = _nan2num(
      jnp.dot(fh_ref[...], w_ref[...], preferred_element_type=jnp.float32)
      + b_ref[...])


def _tc_chol(fh, w_fc, b_fc):
  return pl.pallas_call(
      _tc_chol_body,
      grid=(CHOL_ELEMS // _CB,),
      in_specs=[
          pl.BlockSpec((1, HID), lambda i: (0, 0)),
          pl.BlockSpec((HID, _CB), lambda i: (0, i)),
          pl.BlockSpec((1, _CB), lambda i: (0, i)),
      ],
      out_specs=pl.BlockSpec((1, _CB), lambda i: (0, i)),
      out_shape=jax.ShapeDtypeStruct((1, CHOL_ELEMS), jnp.float32),
  )(fh, w_fc, b_fc.reshape(1, CHOL_ELEMS))


def _tc_llt_body(l_ref, o_ref):
  l = l_ref[...]
  o_ref[...] = _nan2num(
      lax.dot_general(l, l, (((1,), (1,)), ((), ())),
                      preferred_element_type=jnp.float32))


def _tc_llt(l):
  return pl.pallas_call(
      _tc_llt_body,
      out_shape=jax.ShapeDtypeStruct((CHOL_N, CHOL_N), jnp.float32),
  )(l)


def kernel(x, edge_index, edge_weight, W1, b1, W2, b2,
           W_ih, W_hh, b_ih, b_hh, W_fc, b_fc):
  row, col = edge_index[0], edge_index[1]

  npad_e = E_PAD - E_EXT
  loop = jnp.arange(N, dtype=row.dtype)
  pad_r = jnp.arange(npad_e, dtype=row.dtype) % N
  pad_c = N + (jnp.arange(npad_e, dtype=row.dtype) % NDEAD)
  r_ext = jnp.concatenate([row, loop, pad_r])
  c_ext = jnp.concatenate([col, loop, pad_c])
  w_ext = jnp.concatenate(
      [edge_weight,
       jnp.ones((SEQ, N), jnp.float32),
       jnp.zeros((SEQ, npad_e), jnp.float32)], axis=1)

  r2d = r_ext.reshape(NROWS, CHUNK)
  c2d = c_ext.reshape(NROWS, CHUNK)
  w2d = w_ext.reshape(SEQ * NROWS, CHUNK)

  nw2d = _sc_norm(r2d, c2d, w2d)

  x2d = x.reshape(SEQ * N, D)
  xw1 = _tc_mm(x2d, W1)
  p1 = _sc_msg(xw1[0:N], xw1[N:2 * N], xw1[2 * N:3 * N], r2d, c2d, nw2d)
  p1 = p1.reshape(NC, SEQ, NPAD, D)

  xw2 = _tc_fuse_mm(p1[0], p1[1], b1, W2)
  p2 = _sc_msg(xw2[0], xw2[1], xw2[2], r2d, c2d, nw2d)
  p2 = p2.reshape(NC, SEQ, NPAD, D)

  emb = _tc_emb(p2[0], p2[1], b2)
  fh = _tc_lstm(emb, W_ih, W_hh, b_ih, b_hh)
  chol = _tc_chol(fh, W_fc, b_fc)

  chol_pad = jnp.concatenate(
      [chol.reshape(CHOL_ELEMS), jnp.zeros((CHOL_N,), jnp.float32)])
  l = _sc_lbuild(chol_pad)
  return _tc_llt(l)
